# Initial kernel scaffold; baseline (speedup 1.0000x reference)
import jax
import jax.numpy as jnp
from jax.experimental import pallas as pl


def kernel(x, table):
    raise NotImplementedError("write your pallas kernel here")


# speedup vs baseline: 3.5849x; 3.5849x over previous
import jax
import jax.numpy as jnp
from jax import lax
from jax.experimental import pallas as pl
from jax.experimental.pallas import tpu as pltpu
from jax.experimental.pallas import tpu_sc as plsc

_B = 4096 * 200
_D = 30
_DP = 32
_NC, _NS = 2, 16
_NW = _NC * _NS
_BPW = _B // _NW
_CH = 128
_NCHUNK = _BPW // _CH


def _emb_body(x_hbm, table_hbm, out_hbm, idx_v, rows_v, sem):
    wid = lax.axis_index("s") * _NC + lax.axis_index("c")
    base = wid * _BPW

    def body(c, carry):
        off = base + c * _CH
        pltpu.sync_copy(x_hbm.at[pl.ds(off, _CH)], idx_v)
        pltpu.async_copy(table_hbm.at[idx_v], rows_v, sem).wait()
        pltpu.sync_copy(rows_v, out_hbm.at[pl.ds(off, _CH)])
        return carry

    lax.fori_loop(0, _NCHUNK, body, 0)


def kernel(x, table):
    xf = x.reshape(-1)
    tpad = jnp.pad(table, ((0, 0), (0, _DP - _D)))
    mesh = plsc.VectorSubcoreMesh(core_axis_name="c", subcore_axis_name="s")
    f = pl.kernel(
        _emb_body,
        mesh=mesh,
        out_type=jax.ShapeDtypeStruct((_B, _DP), jnp.float32),
        scratch_types=[
            pltpu.VMEM((_CH,), jnp.int32),
            pltpu.VMEM((_CH, _DP), jnp.float32),
            pltpu.SemaphoreType.DMA,
        ],
        compiler_params=pltpu.CompilerParams(use_tc_tiling_on_sc=False),
    )
    out = f(xf, tpad)
    return out[:, :_D].reshape(x.shape[0], x.shape[1], _D)


# speedup vs baseline: 4.9893x; 1.3918x over previous
import jax
import jax.numpy as jnp
from jax import lax
from jax.experimental import pallas as pl
from jax.experimental.pallas import tpu as pltpu
from jax.experimental.pallas import tpu_sc as plsc

_B = 4096 * 200
_D = 30
_DP = 32
_NC, _NS = 2, 16
_NW = _NC * _NS
_BPW = _B // _NW
_CH = 512
_NCHUNK = _BPW // _CH
_NSTEP = _NCHUNK // 2


def _emb_body(x_hbm, table_hbm, out_hbm, idx_all,
              rows0, rows1, gsem0, gsem1, wsem0, wsem1):
    wid = lax.axis_index("s") * _NC + lax.axis_index("c")
    base = wid * _BPW
    rows = (rows0, rows1)
    gsem = (gsem0, gsem1)
    wsem = (wsem0, wsem1)

    pltpu.sync_copy(x_hbm.at[pl.ds(base, _BPW)], idx_all)

    def g_start(c, b):
        pltpu.async_copy(
            table_hbm.at[idx_all.at[pl.ds(c * _CH, _CH)]], rows[b], gsem[b])

    def g_wait(b):
        pltpu.make_async_copy(
            table_hbm.at[idx_all.at[pl.ds(0, _CH)]], rows[b], gsem[b]).wait()

    def w_start(c, b):
        pltpu.async_copy(
            rows[b], out_hbm.at[pl.ds(base + c * _CH, _CH)], wsem[b])

    def w_wait(b):
        pltpu.make_async_copy(
            rows[b], out_hbm.at[pl.ds(base, _CH)], wsem[b]).wait()

    g_start(0, 0)
    g_start(1, 1)

    def body(p, carry):
        for b in range(2):
            c = 2 * p + b
            g_wait(b)
            w_start(c, b)
            w_wait(b)

            @pl.when(c + 2 < _NCHUNK)
            def _():
                g_start(c + 2, b)

        return carry

    lax.fori_loop(0, _NSTEP, body, 0)


def kernel(x, table):
    xf = x.reshape(-1)
    tpad = jnp.pad(table, ((0, 0), (0, _DP - _D)))
    mesh = plsc.VectorSubcoreMesh(core_axis_name="c", subcore_axis_name="s")
    f = pl.kernel(
        _emb_body,
        mesh=mesh,
        out_type=jax.ShapeDtypeStruct((_B, _DP), jnp.float32),
        scratch_types=[
            pltpu.VMEM((_BPW,), jnp.int32),
            pltpu.VMEM((_CH, _DP), jnp.float32),
            pltpu.VMEM((_CH, _DP), jnp.float32),
            pltpu.SemaphoreType.DMA,
            pltpu.SemaphoreType.DMA,
            pltpu.SemaphoreType.DMA,
            pltpu.SemaphoreType.DMA,
        ],
        compiler_params=pltpu.CompilerParams(use_tc_tiling_on_sc=False),
    )
    out = f(xf, tpad)
    return out[:, :_D].reshape(x.shape[0], x.shape[1], _D)
